# initial kernel scaffold (unmeasured)
import jax
import jax.numpy as jnp
from jax import lax
from jax.experimental import pallas as pl
from jax.experimental.pallas import tpu as pltpu

D = 4096
M_SHARD = 4096
CH = 512
NC = M_SHARD // CH
EPS = 1e-6


def kernel(partial, gamma):
    gamma2 = gamma.reshape(1, D)

    def body(p_ref, g_ref, out_ref, send_buf, recv_buf, stage, out_stage,
             send_sem, recv_sem, local_sem, out_sem):
        my_x = lax.axis_index("x")
        my_y = lax.axis_index("y")
        my_z = lax.axis_index("z")
        peer = (my_x, 1 - my_y, my_z)

        barrier = pltpu.get_barrier_semaphore()
        pl.semaphore_signal(barrier, inc=1, device_id=peer,
                            device_id_type=pl.DeviceIdType.MESH)
        pl.semaphore_wait(barrier, 1)

        peer_row0 = (1 - my_y) * M_SHARD
        my_row0 = my_y * M_SHARD

        for c in range(NC):
            cp = pltpu.make_async_copy(
                p_ref.at[0, pl.ds(peer_row0 + c * CH, CH), :],
                stage, local_sem)
            cp.start()
            cp.wait()
            send_buf[pl.ds(c * CH, CH), :] = stage[:, :].astype(jnp.bfloat16)

        rdma = pltpu.make_async_remote_copy(
            src_ref=send_buf, dst_ref=recv_buf,
            send_sem=send_sem, recv_sem=recv_sem,
            device_id=peer, device_id_type=pl.DeviceIdType.MESH)
        rdma.start()
        rdma.wait()

        for c in range(NC):
            cp = pltpu.make_async_copy(
                p_ref.at[0, pl.ds(my_row0 + c * CH, CH), :],
                stage, local_sem)
            cp.start()
            cp.wait()
            s = stage[:, :] + recv_buf[pl.ds(c * CH, CH), :].astype(jnp.float32)
            r = lax.rsqrt(jnp.mean(s * s, axis=1, keepdims=True) + EPS)
            out_stage[:, :] = s * r * g_ref[:, :]
            st = pltpu.make_async_copy(
                out_stage, out_ref.at[pl.ds(c * CH, CH), :], out_sem)
            st.start()
            st.wait()

    return pl.pallas_call(
        body,
        out_shape=jax.ShapeDtypeStruct((M_SHARD, D), jnp.float32),
        in_specs=[
            pl.BlockSpec(memory_space=pltpu.ANY),
            pl.BlockSpec(memory_space=pltpu.VMEM),
        ],
        out_specs=pl.BlockSpec(memory_space=pltpu.ANY),
        scratch_shapes=[
            pltpu.VMEM((M_SHARD, D), jnp.bfloat16),
            pltpu.VMEM((M_SHARD, D), jnp.bfloat16),
            pltpu.VMEM((CH, D), jnp.float32),
            pltpu.VMEM((CH, D), jnp.float32),
            pltpu.SemaphoreType.DMA,
            pltpu.SemaphoreType.DMA,
            pltpu.SemaphoreType.DMA,
            pltpu.SemaphoreType.DMA,
        ],
        compiler_params=pltpu.CompilerParams(collective_id=0),
    )(partial, gamma2)


# baseline (device time: 452634 ns/iter reference)
import jax
import jax.numpy as jnp
from jax import lax
from jax.experimental import pallas as pl
from jax.experimental.pallas import tpu as pltpu

D = 4096
M_SHARD = 4096
CH = 256
NC = M_SHARD // CH
NSEND = 2
EPS = 1e-6


def kernel(partial, gamma):
    gamma2 = gamma.reshape(1, D)

    def body(p_ref, g_ref, out_ref, send_buf, recv_buf, stage, out_stage,
             send_sems, recv_sems, local_sem, out_sem):
        my_x = lax.axis_index("x")
        my_y = lax.axis_index("y")
        my_z = lax.axis_index("z")
        peer = (my_x, 1 - my_y, my_z)

        barrier = pltpu.get_barrier_semaphore()
        pl.semaphore_signal(barrier, inc=1, device_id=peer,
                            device_id_type=pl.DeviceIdType.MESH)
        pl.semaphore_wait(barrier, 1)

        peer_row0 = (1 - my_y) * M_SHARD
        my_row0 = my_y * M_SHARD

        rdmas = []
        for c in range(NC):
            slot = c % NSEND
            if c >= NSEND:
                rdmas[c - NSEND].wait_send()
            cp = pltpu.make_async_copy(
                p_ref.at[0, pl.ds(peer_row0 + c * CH, CH), :],
                stage, local_sem)
            cp.start()
            cp.wait()
            send_buf[slot] = stage[:, :].astype(jnp.bfloat16)
            rdma = pltpu.make_async_remote_copy(
                src_ref=send_buf.at[slot], dst_ref=recv_buf.at[c],
                send_sem=send_sems.at[slot], recv_sem=recv_sems.at[c],
                device_id=peer, device_id_type=pl.DeviceIdType.MESH)
            rdma.start()
            rdmas.append(rdma)

        for c in range(NC):
            cp = pltpu.make_async_copy(
                p_ref.at[0, pl.ds(my_row0 + c * CH, CH), :],
                stage, local_sem)
            cp.start()
            cp.wait()
            rdmas[c].wait_recv()
            s = stage[:, :] + recv_buf[c].astype(jnp.float32)
            r = lax.rsqrt(jnp.mean(s * s, axis=1, keepdims=True) + EPS)
            out_stage[:, :] = s * r * g_ref[:, :]
            st = pltpu.make_async_copy(
                out_stage, out_ref.at[pl.ds(c * CH, CH), :], out_sem)
            st.start()
            st.wait()

        for c in range(NC - NSEND, NC):
            rdmas[c].wait_send()

    return pl.pallas_call(
        body,
        out_shape=jax.ShapeDtypeStruct((M_SHARD, D), jnp.float32),
        in_specs=[
            pl.BlockSpec(memory_space=pl.ANY),
            pl.BlockSpec(memory_space=pltpu.VMEM),
        ],
        out_specs=pl.BlockSpec(memory_space=pl.ANY),
        scratch_shapes=[
            pltpu.VMEM((NSEND, CH, D), jnp.bfloat16),
            pltpu.VMEM((NC, CH, D), jnp.bfloat16),
            pltpu.VMEM((CH, D), jnp.float32),
            pltpu.VMEM((CH, D), jnp.float32),
            pltpu.SemaphoreType.DMA((NSEND,)),
            pltpu.SemaphoreType.DMA((NC,)),
            pltpu.SemaphoreType.DMA,
            pltpu.SemaphoreType.DMA,
        ],
        compiler_params=pltpu.CompilerParams(
            collective_id=0, vmem_limit_bytes=60 * 1024 * 1024),
    )(partial, gamma2)


# device time: 376225 ns/iter; 1.2031x vs baseline; 1.2031x over previous
import jax
import jax.numpy as jnp
from jax import lax
from jax.experimental import pallas as pl
from jax.experimental.pallas import tpu as pltpu

D = 4096
M_SHARD = 4096
OWN = M_SHARD // 2
CH = 256
NC = OWN // CH
NSEND = 2
EPS = 1e-6


def kernel(partial, gamma):
    gamma2 = gamma.reshape(1, D)

    def body(p_ref, g_ref, out_ref,
             y_send, y_recv, x_send, x_recv, stage, out_stage,
             y_send_sems, y_recv_sems, x_send_sems, x_recv_sems,
             local_sems, out_sem):
        my_x = lax.axis_index("x")
        my_y = lax.axis_index("y")
        my_z = lax.axis_index("z")
        y_peer = (my_x, 1 - my_y, my_z)
        x_peer = (1 - my_x, my_y, my_z)

        barrier = pltpu.get_barrier_semaphore()
        for nbr in (y_peer, x_peer):
            pl.semaphore_signal(barrier, inc=1, device_id=nbr,
                                device_id_type=pl.DeviceIdType.MESH)
        pl.semaphore_wait(barrier, 2)

        ysend_row0 = (1 - my_y) * M_SHARD + my_x * OWN
        mine_row0 = my_y * M_SHARD + my_x * OWN
        out_mine = my_x * OWN
        out_theirs = (1 - my_x) * OWN

        y_rdmas = []
        for c in range(NC):
            slot = c % NSEND
            if c >= NSEND:
                y_rdmas[c - NSEND].wait_send()
            cp = pltpu.make_async_copy(
                p_ref.at[0, pl.ds(ysend_row0 + c * CH, CH), :],
                stage.at[0], local_sems.at[0])
            cp.start()
            cp.wait()
            y_send[slot] = stage[0].astype(jnp.bfloat16)
            rdma = pltpu.make_async_remote_copy(
                src_ref=y_send.at[slot], dst_ref=y_recv.at[c],
                send_sem=y_send_sems.at[slot], recv_sem=y_recv_sems.at[c],
                device_id=y_peer, device_id_type=pl.DeviceIdType.MESH)
            rdma.start()
            y_rdmas.append(rdma)

        cp = pltpu.make_async_copy(
            p_ref.at[0, pl.ds(mine_row0, CH), :],
            stage.at[0], local_sems.at[0])
        cp.start()
        x_rdmas = []
        for c in range(NC):
            if c + 1 < NC:
                nxt = pltpu.make_async_copy(
                    p_ref.at[0, pl.ds(mine_row0 + (c + 1) * CH, CH), :],
                    stage.at[(c + 1) % 2], local_sems.at[(c + 1) % 2])
                nxt.start()
            pltpu.make_async_copy(
                p_ref.at[0, pl.ds(mine_row0 + c * CH, CH), :],
                stage.at[c % 2], local_sems.at[c % 2]).wait()
            y_rdmas[c].wait_recv()
            s = stage[c % 2] + y_recv[c].astype(jnp.float32)
            r = lax.rsqrt(jnp.mean(s * s, axis=1, keepdims=True) + EPS)
            out_stage[:, :] = s * r * g_ref[:, :]
            slot = c % NSEND
            if c >= NSEND:
                x_rdmas[c - NSEND].wait_send()
            x_send[slot] = out_stage[:, :].astype(jnp.bfloat16)
            rdma = pltpu.make_async_remote_copy(
                src_ref=x_send.at[slot], dst_ref=x_recv.at[c],
                send_sem=x_send_sems.at[slot], recv_sem=x_recv_sems.at[c],
                device_id=x_peer, device_id_type=pl.DeviceIdType.MESH)
            rdma.start()
            x_rdmas.append(rdma)
            st = pltpu.make_async_copy(
                out_stage, out_ref.at[pl.ds(out_mine + c * CH, CH), :],
                out_sem)
            st.start()
            st.wait()

        for c in range(NC):
            x_rdmas[c].wait_recv()
            out_stage[:, :] = x_recv[c].astype(jnp.float32)
            st = pltpu.make_async_copy(
                out_stage, out_ref.at[pl.ds(out_theirs + c * CH, CH), :],
                out_sem)
            st.start()
            st.wait()

        for c in range(NC - NSEND, NC):
            y_rdmas[c].wait_send()
            x_rdmas[c].wait_send()

    return pl.pallas_call(
        body,
        out_shape=jax.ShapeDtypeStruct((M_SHARD, D), jnp.float32),
        in_specs=[
            pl.BlockSpec(memory_space=pl.ANY),
            pl.BlockSpec(memory_space=pltpu.VMEM),
        ],
        out_specs=pl.BlockSpec(memory_space=pl.ANY),
        scratch_shapes=[
            pltpu.VMEM((NSEND, CH, D), jnp.bfloat16),
            pltpu.VMEM((NC, CH, D), jnp.bfloat16),
            pltpu.VMEM((NSEND, CH, D), jnp.bfloat16),
            pltpu.VMEM((NC, CH, D), jnp.bfloat16),
            pltpu.VMEM((2, CH, D), jnp.float32),
            pltpu.VMEM((CH, D), jnp.float32),
            pltpu.SemaphoreType.DMA((NSEND,)),
            pltpu.SemaphoreType.DMA((NC,)),
            pltpu.SemaphoreType.DMA((NSEND,)),
            pltpu.SemaphoreType.DMA((NC,)),
            pltpu.SemaphoreType.DMA((2,)),
            pltpu.SemaphoreType.DMA,
        ],
        compiler_params=pltpu.CompilerParams(
            collective_id=0, vmem_limit_bytes=60 * 1024 * 1024),
    )(partial, gamma2)


# device time: 260938 ns/iter; 1.7346x vs baseline; 1.4418x over previous
import jax
import jax.numpy as jnp
from jax import lax
from jax.experimental import pallas as pl
from jax.experimental.pallas import tpu as pltpu

D = 4096
M_SHARD = 4096
OWN = M_SHARD // 2
CH = 256
NC = OWN // CH
NSEND = 2
EPS = 1e-6


def kernel(partial, gamma):
    gamma2 = gamma.reshape(1, D)

    def body(p_ref, g_ref, out_ref,
             y_send, y_recv, x_send, x_recv, fill_stage, comp_stage,
             out_stage,
             y_send_sems, y_recv_sems, x_send_sems, x_recv_sems,
             fill_sem, comp_sems, out_sem):
        my_x = lax.axis_index("x")
        my_y = lax.axis_index("y")
        my_z = lax.axis_index("z")
        y_peer = (my_x, 1 - my_y, my_z)
        x_peer = (1 - my_x, my_y, my_z)

        barrier = pltpu.get_barrier_semaphore()
        for nbr in (y_peer, x_peer):
            pl.semaphore_signal(barrier, inc=1, device_id=nbr,
                                device_id_type=pl.DeviceIdType.MESH)
        pl.semaphore_wait(barrier, 2)

        ysend_row0 = (1 - my_y) * M_SHARD + my_x * OWN
        mine_row0 = my_y * M_SHARD + my_x * OWN
        out_mine = my_x * OWN
        out_theirs = (1 - my_x) * OWN

        y_rdmas = []
        x_rdmas = []

        def comp_dma(c):
            return pltpu.make_async_copy(
                p_ref.at[0, pl.ds(mine_row0 + c * CH, CH), :],
                comp_stage.at[c % 2], comp_sems.at[c % 2])

        def fill_and_send_y(c):
            slot = c % NSEND
            if c >= NSEND:
                y_rdmas[c - NSEND].wait_send()
            cp = pltpu.make_async_copy(
                p_ref.at[0, pl.ds(ysend_row0 + c * CH, CH), :],
                fill_stage, fill_sem)
            cp.start()
            cp.wait()
            y_send[slot] = fill_stage[:, :].astype(jnp.bfloat16)
            rdma = pltpu.make_async_remote_copy(
                src_ref=y_send.at[slot], dst_ref=y_recv.at[c],
                send_sem=y_send_sems.at[slot], recv_sem=y_recv_sems.at[c],
                device_id=y_peer, device_id_type=pl.DeviceIdType.MESH)
            rdma.start()
            y_rdmas.append(rdma)

        def process(c):
            comp_dma(c).wait()
            y_rdmas[c].wait_recv()
            s = comp_stage[c % 2] + y_recv[c].astype(jnp.float32)
            r = lax.rsqrt(jnp.mean(s * s, axis=1, keepdims=True) + EPS)
            out_stage[:, :] = s * r * g_ref[:, :]
            slot = c % NSEND
            if c >= NSEND:
                x_rdmas[c - NSEND].wait_send()
            x_send[slot] = out_stage[:, :].astype(jnp.bfloat16)
            rdma = pltpu.make_async_remote_copy(
                src_ref=x_send.at[slot], dst_ref=x_recv.at[c],
                send_sem=x_send_sems.at[slot], recv_sem=x_recv_sems.at[c],
                device_id=x_peer, device_id_type=pl.DeviceIdType.MESH)
            rdma.start()
            x_rdmas.append(rdma)
            st = pltpu.make_async_copy(
                out_stage, out_ref.at[pl.ds(out_mine + c * CH, CH), :],
                out_sem)
            st.start()
            st.wait()

        comp_dma(0).start()
        fill_and_send_y(0)
        for c in range(1, NC):
            comp_dma(c).start()
            fill_and_send_y(c)
            process(c - 1)
        process(NC - 1)

        for c in range(NC):
            x_rdmas[c].wait_recv()
            out_stage[:, :] = x_recv[c].astype(jnp.float32)
            st = pltpu.make_async_copy(
                out_stage, out_ref.at[pl.ds(out_theirs + c * CH, CH), :],
                out_sem)
            st.start()
            st.wait()

        for c in range(NC - NSEND, NC):
            y_rdmas[c].wait_send()
            x_rdmas[c].wait_send()

    return pl.pallas_call(
        body,
        out_shape=jax.ShapeDtypeStruct((M_SHARD, D), jnp.float32),
        in_specs=[
            pl.BlockSpec(memory_space=pl.ANY),
            pl.BlockSpec(memory_space=pltpu.VMEM),
        ],
        out_specs=pl.BlockSpec(memory_space=pl.ANY),
        scratch_shapes=[
            pltpu.VMEM((NSEND, CH, D), jnp.bfloat16),
            pltpu.VMEM((NC, CH, D), jnp.bfloat16),
            pltpu.VMEM((NSEND, CH, D), jnp.bfloat16),
            pltpu.VMEM((NC, CH, D), jnp.bfloat16),
            pltpu.VMEM((CH, D), jnp.float32),
            pltpu.VMEM((2, CH, D), jnp.float32),
            pltpu.VMEM((CH, D), jnp.float32),
            pltpu.SemaphoreType.DMA((NSEND,)),
            pltpu.SemaphoreType.DMA((NC,)),
            pltpu.SemaphoreType.DMA((NSEND,)),
            pltpu.SemaphoreType.DMA((NC,)),
            pltpu.SemaphoreType.DMA,
            pltpu.SemaphoreType.DMA((2,)),
            pltpu.SemaphoreType.DMA,
        ],
        compiler_params=pltpu.CompilerParams(
            collective_id=0, vmem_limit_bytes=62 * 1024 * 1024),
    )(partial, gamma2)
